# baseline (speedup 1.0000x reference)
import jax
import jax.numpy as jnp
from jax import lax
from jax.experimental import pallas as pl
from jax.experimental.pallas import tpu as pltpu
from jax.experimental.pallas import tpu_sc as plsc

_N = 10000
_E = 320000
_D = 128
_H = 64
_C = 64
_ALPHA = 0.1
_K = 10

_NC = 2
_NS = 16
_NW = _NC * _NS
_EPT = _E // _NW
_G = 128
_NCHUNK = 79
_EPAD = _NW * _NCHUNK * _G
_NP = 10112
_RPT = _NP // _NS
_SB = 1000

_ROWBLK = 1000


def _mlp_block(x_ref, w1_ref, b1_ref, w2_ref, b2_ref, z_ref, z9_ref):
    h = jnp.dot(x_ref[...], w1_ref[...], preferred_element_type=jnp.float32)
    h = jnp.maximum(h + b1_ref[...], 0.0)
    z = jnp.dot(h, w2_ref[...], preferred_element_type=jnp.float32) + b2_ref[...]
    z_ref[...] = z
    z9_ref[...] = z * (_ALPHA / (1.0 - _ALPHA))


def _final_block(a0_ref, a1_ref, o_ref):
    xk = (1.0 - _ALPHA) * (a0_ref[0] + a1_ref[0])
    m = jnp.max(xk, axis=1, keepdims=True)
    s = jnp.sum(jnp.exp(xk - m), axis=1, keepdims=True)
    o_ref[...] = (xk - m) - jnp.log(s)


def _prop_step_body(pp_hbm, z9_hbm, src_hbm, dst_hbm, w_hbm, zeros_hbm,
                    part_hbm, acc, xk_s, src_v, dst_v, w_v,
                    rows0, rows1, sbuf0, sbuf1, stg_a, stg_b, stg_o,
                    gsem0, gsem1, ssem0, ssem1):
    cid = lax.axis_index("c")
    sid = lax.axis_index("s")
    gid = cid * _NS + sid

    @pl.when(cid == 0)
    def _():
        pltpu.sync_copy(z9_hbm.at[pl.ds(sid * _RPT, _RPT)],
                        acc.at[pl.ds(sid * _RPT, _RPT)])

    @pl.when(cid == 1)
    def _():
        pltpu.sync_copy(zeros_hbm, acc.at[pl.ds(sid * _RPT, _RPT)])

    def blend_block(base, nrows):
        pltpu.async_copy(pp_hbm.at[0, pl.ds(base, nrows)],
                         stg_a.at[pl.ds(0, nrows)], gsem0)
        pltpu.async_copy(pp_hbm.at[1, pl.ds(base, nrows)],
                         stg_b.at[pl.ds(0, nrows)], gsem1)
        pltpu.make_async_copy(pp_hbm.at[0, pl.ds(base, nrows)],
                              stg_a.at[pl.ds(0, nrows)], gsem0).wait()
        pltpu.make_async_copy(pp_hbm.at[1, pl.ds(base, nrows)],
                              stg_b.at[pl.ds(0, nrows)], gsem1).wait()

        def rowloop(rr, c2):
            for half in range(2):
                sl0 = pl.ds(half * 32, 16)
                sl1 = pl.ds(half * 32 + 16, 16)
                va = (1.0 - _ALPHA) * (stg_a[rr, sl0] + stg_b[rr, sl0])
                vb = (1.0 - _ALPHA) * (stg_a[rr, sl1] + stg_b[rr, sl1])
                stg_o[rr, pl.ds(half * 32, 32)] = plsc.pack(
                    va, vb, format=plsc.PackFormat.INTERLEAVED)
            return c2

        lax.fori_loop(0, nrows, rowloop, 0)
        pltpu.sync_copy(stg_o.at[pl.ds(0, nrows)], xk_s.at[pl.ds(base, nrows)])

    @pl.when(sid < 10)
    def _():
        def stage_blk(b, carry):
            blend_block(sid * _SB + b * 80, 80)
            return carry

        lax.fori_loop(0, _SB // 80, stage_blk, 0)
        blend_block(sid * _SB + (_SB // 80) * 80, _SB % 80)

    pltpu.sync_copy(src_hbm.at[gid], src_v)
    pltpu.sync_copy(dst_hbm.at[gid], dst_v)
    pltpu.sync_copy(w_hbm.at[gid], w_v)
    plsc.subcore_barrier()

    rows = (rows0, rows1)
    sbufs = (sbuf0, sbuf1)
    gsems = (gsem0, gsem1)
    ssems = (ssem0, ssem1)

    def scale(src_buf, dst_buf, j):
        for r in range(_G // 16):
            w16 = w_v[j, pl.ds(r * 16, 16)]
            for e in range(16):
                wv = w16[e]
                row = r * 16 + e
                for half in range(2):
                    ab = src_buf[row, pl.ds(half * 32, 32)]
                    va, vb = plsc.unpack(ab, format=plsc.PackFormat.INTERLEAVED)
                    dst_buf[row, pl.ds(half * 32, 16)] = va * wv
                    dst_buf[row, pl.ds(half * 32 + 16, 16)] = vb * wv

    pltpu.async_copy(xk_s.at[src_v.at[0]], rows0, gsem0)

    def body(t, carry):
        for p in range(2):
            i = 2 * t + p
            buf, sbuf = rows[p], sbufs[p]

            @pl.when(i < _NCHUNK)
            def _():
                pltpu.make_async_copy(xk_s.at[src_v.at[i]], buf,
                                      gsems[p]).wait()

                @pl.when(i < _NCHUNK - 1)
                def _():
                    pltpu.async_copy(xk_s.at[src_v.at[i + 1]], rows[1 - p],
                                     gsems[1 - p])

                @pl.when(i >= 2)
                def _():
                    pltpu.make_async_copy(sbuf, acc.at[dst_v.at[0]],
                                          ssems[p]).wait()

                scale(buf, sbuf, i)
                pltpu.async_copy(sbuf, acc.at[dst_v.at[i]], ssems[p],
                                 add=True)
        return carry

    lax.fori_loop(0, (_NCHUNK + 1) // 2, body, 0)
    pltpu.make_async_copy(sbuf1, acc.at[dst_v.at[0]], ssem1).wait()
    pltpu.make_async_copy(sbuf0, acc.at[dst_v.at[0]], ssem0).wait()

    plsc.subcore_barrier()
    pltpu.sync_copy(acc.at[pl.ds(sid * _RPT, _RPT)],
                    part_hbm.at[cid, pl.ds(sid * _RPT, _RPT)])


def _make_prop_step():
    mesh = plsc.VectorSubcoreMesh(core_axis_name="c", subcore_axis_name="s")
    return pl.kernel(
        _prop_step_body,
        out_type=jax.ShapeDtypeStruct((_NC, _NP, _C), jnp.float32),
        mesh=mesh,
        scratch_types=[
            pltpu.VMEM_SHARED((_NP, _C), jnp.float32),
            pltpu.VMEM_SHARED((_N, _C), jnp.bfloat16),
            pltpu.VMEM((_NCHUNK, _G), jnp.int32),
            pltpu.VMEM((_NCHUNK, _G), jnp.int32),
            pltpu.VMEM((_NCHUNK, _G), jnp.float32),
            pltpu.VMEM((_G, _C), jnp.bfloat16),
            pltpu.VMEM((_G, _C), jnp.bfloat16),
            pltpu.VMEM((_G, _C), jnp.float32),
            pltpu.VMEM((_G, _C), jnp.float32),
            pltpu.VMEM((80, _C), jnp.float32),
            pltpu.VMEM((80, _C), jnp.float32),
            pltpu.VMEM((80, _C), jnp.bfloat16),
            pltpu.SemaphoreType.DMA,
            pltpu.SemaphoreType.DMA,
            pltpu.SemaphoreType.DMA,
            pltpu.SemaphoreType.DMA,
        ],
        compiler_params=pltpu.CompilerParams(use_tc_tiling_on_sc=False,
                                             needs_layout_passes=False),
    )


def kernel(x, edge_index, edge_weight, W1, b1, W2, b2):
    npad = _EPAD - _E
    izero = jnp.zeros((npad,), jnp.int32)
    dst = jnp.concatenate([edge_index[0], izero]).reshape(_NW, _NCHUNK, _G)
    src = jnp.concatenate([edge_index[1], izero]).reshape(_NW, _NCHUNK, _G)
    w = jnp.concatenate([edge_weight, jnp.zeros((npad,), jnp.float32)]
                        ).reshape(_NW, _NCHUNK, _G)
    zeros = jnp.zeros((_RPT, _C), jnp.float32)

    mlp = pl.pallas_call(
        _mlp_block,
        grid=(_N // _ROWBLK,),
        in_specs=[
            pl.BlockSpec((_ROWBLK, _D), lambda i: (i, 0)),
            pl.BlockSpec((_D, _H), lambda i: (0, 0)),
            pl.BlockSpec((1, _H), lambda i: (0, 0)),
            pl.BlockSpec((_H, _C), lambda i: (0, 0)),
            pl.BlockSpec((1, _C), lambda i: (0, 0)),
        ],
        out_specs=[
            pl.BlockSpec((_ROWBLK, _C), lambda i: (i, 0)),
            pl.BlockSpec((_ROWBLK, _C), lambda i: (i, 0)),
        ],
        out_shape=[
            jax.ShapeDtypeStruct((_N, _C), jnp.float32),
            jax.ShapeDtypeStruct((_NP, _C), jnp.float32),
        ],
    )
    z, z9 = mlp(x, W1, b1.reshape(1, _H), W2, b2.reshape(1, _C))

    step = _make_prop_step()

    final = pl.pallas_call(
        _final_block,
        grid=(_N // _ROWBLK,),
        in_specs=[
            pl.BlockSpec((1, _ROWBLK, _C), lambda i: (0, i, 0)),
            pl.BlockSpec((1, _ROWBLK, _C), lambda i: (1, i, 0)),
        ],
        out_specs=pl.BlockSpec((_ROWBLK, _C), lambda i: (i, 0)),
        out_shape=jax.ShapeDtypeStruct((_N, _C), jnp.float32),
    )

    part = jnp.zeros((_NC, _NP, _C), jnp.float32)
    part = part.at[0, :_N].set(z * (1.0 / (1.0 - _ALPHA)))
    for _ in range(_K):
        part = step(part, z9, src, dst, w, zeros)
    return final(part, part)


# speedup vs baseline: 1.1008x; 1.1008x over previous
import jax
import jax.numpy as jnp
from jax import lax
from jax.experimental import pallas as pl
from jax.experimental.pallas import tpu as pltpu
from jax.experimental.pallas import tpu_sc as plsc

_N = 10000
_E = 320000
_D = 128
_H = 64
_C = 64
_ALPHA = 0.1
_K = 10

_NC = 2
_NS = 16
_NW = _NC * _NS
_EPT = _E // _NW
_G = 80
_NCHUNK = _EPT // _G
_NP = 10112
_RPT = _NP // _NS
_SB = 1000

_ROWBLK = 1000


def _mlp_block(x_ref, w1_ref, b1_ref, w2_ref, b2_ref, z_ref, z9_ref):
    h = jnp.dot(x_ref[...], w1_ref[...], preferred_element_type=jnp.float32)
    h = jnp.maximum(h + b1_ref[...], 0.0)
    z = jnp.dot(h, w2_ref[...], preferred_element_type=jnp.float32) + b2_ref[...]
    z_ref[...] = z
    z9_ref[...] = z * (_ALPHA / (1.0 - _ALPHA))


def _final_block(a0_ref, a1_ref, o_ref):
    xk = (1.0 - _ALPHA) * (a0_ref[0] + a1_ref[0])
    m = jnp.max(xk, axis=1, keepdims=True)
    s = jnp.sum(jnp.exp(xk - m), axis=1, keepdims=True)
    o_ref[...] = (xk - m) - jnp.log(s)


def _prop_step_body(pp_hbm, z9_hbm, src_hbm, dst_hbm, w_hbm, zeros_hbm,
                    part_hbm, acc, xk_s, src_v, dst_v, w_v,
                    rows0, rows1, rows2, rows3, sbuf0, sbuf1,
                    stg_a, stg_b, stg_o,
                    gsem0, gsem1, gsem2, gsem3, ssem0, ssem1):
    cid = lax.axis_index("c")
    sid = lax.axis_index("s")
    gid = cid * _NS + sid

    @pl.when(cid == 0)
    def _():
        pltpu.sync_copy(z9_hbm.at[pl.ds(sid * _RPT, _RPT)],
                        acc.at[pl.ds(sid * _RPT, _RPT)])

    @pl.when(cid == 1)
    def _():
        pltpu.sync_copy(zeros_hbm, acc.at[pl.ds(sid * _RPT, _RPT)])

    def blend_block(base, nrows):
        pltpu.async_copy(pp_hbm.at[0, pl.ds(base, nrows)],
                         stg_a.at[pl.ds(0, nrows)], gsem0)
        pltpu.async_copy(pp_hbm.at[1, pl.ds(base, nrows)],
                         stg_b.at[pl.ds(0, nrows)], gsem1)
        pltpu.make_async_copy(pp_hbm.at[0, pl.ds(base, nrows)],
                              stg_a.at[pl.ds(0, nrows)], gsem0).wait()
        pltpu.make_async_copy(pp_hbm.at[1, pl.ds(base, nrows)],
                              stg_b.at[pl.ds(0, nrows)], gsem1).wait()

        def rowloop(rr, c2):
            for half in range(2):
                sl0 = pl.ds(half * 32, 16)
                sl1 = pl.ds(half * 32 + 16, 16)
                va = (1.0 - _ALPHA) * (stg_a[rr, sl0] + stg_b[rr, sl0])
                vb = (1.0 - _ALPHA) * (stg_a[rr, sl1] + stg_b[rr, sl1])
                stg_o[rr, pl.ds(half * 32, 32)] = plsc.pack(
                    va, vb, format=plsc.PackFormat.INTERLEAVED)
            return c2

        lax.fori_loop(0, nrows, rowloop, 0)
        pltpu.sync_copy(stg_o.at[pl.ds(0, nrows)], xk_s.at[pl.ds(base, nrows)])

    @pl.when(sid < 10)
    def _():
        def stage_blk(b, carry):
            blend_block(sid * _SB + b * _G, _G)
            return carry

        lax.fori_loop(0, _SB // _G, stage_blk, 0)
        blend_block(sid * _SB + (_SB // _G) * _G, _SB % _G)

    pltpu.sync_copy(src_hbm.at[gid], src_v)
    pltpu.sync_copy(dst_hbm.at[gid], dst_v)
    pltpu.sync_copy(w_hbm.at[gid], w_v)
    plsc.subcore_barrier()

    rows = (rows0, rows1, rows2, rows3)
    sbufs = (sbuf0, sbuf1)
    gsems = (gsem0, gsem1, gsem2, gsem3)
    ssems = (ssem0, ssem1)

    def scale(src_buf, dst_buf, j):
        for r in range(_G // 16):
            w16 = w_v[j, pl.ds(r * 16, 16)]
            for e in range(16):
                wv = w16[e]
                row = r * 16 + e
                for half in range(2):
                    ab = src_buf[row, pl.ds(half * 32, 32)]
                    va, vb = plsc.unpack(ab, format=plsc.PackFormat.INTERLEAVED)
                    dst_buf[row, pl.ds(half * 32, 16)] = va * wv
                    dst_buf[row, pl.ds(half * 32 + 16, 16)] = vb * wv

    pltpu.async_copy(xk_s.at[src_v.at[0]], rows0, gsem0)
    pltpu.async_copy(xk_s.at[src_v.at[1]], rows1, gsem1)

    def body(t, carry):
        for p in range(4):
            i = 4 * t + p
            buf, sbuf = rows[p], sbufs[p % 2]
            np_, ngsem = rows[(p + 2) % 4], gsems[(p + 2) % 4]

            @pl.when(i < _NCHUNK)
            def _():
                pltpu.make_async_copy(xk_s.at[src_v.at[i]], buf,
                                      gsems[p]).wait()

                @pl.when(i + 2 < _NCHUNK)
                def _():
                    pltpu.async_copy(xk_s.at[src_v.at[i + 2]], np_, ngsem)

                @pl.when(i >= 2)
                def _():
                    pltpu.make_async_copy(sbuf, acc.at[dst_v.at[0]],
                                          ssems[p % 2]).wait()

                scale(buf, sbuf, i)
                pltpu.async_copy(sbuf, acc.at[dst_v.at[i]], ssems[p % 2],
                                 add=True)
        return carry

    lax.fori_loop(0, (_NCHUNK + 3) // 4, body, 0)
    pltpu.make_async_copy(sbuf1, acc.at[dst_v.at[0]], ssem1).wait()
    pltpu.make_async_copy(sbuf0, acc.at[dst_v.at[0]], ssem0).wait()

    plsc.subcore_barrier()
    pltpu.sync_copy(acc.at[pl.ds(sid * _RPT, _RPT)],
                    part_hbm.at[cid, pl.ds(sid * _RPT, _RPT)])


def _make_prop_step():
    mesh = plsc.VectorSubcoreMesh(core_axis_name="c", subcore_axis_name="s")
    return pl.kernel(
        _prop_step_body,
        out_type=jax.ShapeDtypeStruct((_NC, _NP, _C), jnp.float32),
        mesh=mesh,
        scratch_types=[
            pltpu.VMEM_SHARED((_NP, _C), jnp.float32),
            pltpu.VMEM_SHARED((_N, _C), jnp.bfloat16),
            pltpu.VMEM((_NCHUNK, _G), jnp.int32),
            pltpu.VMEM((_NCHUNK, _G), jnp.int32),
            pltpu.VMEM((_NCHUNK, _G), jnp.float32),
            pltpu.VMEM((_G, _C), jnp.bfloat16),
            pltpu.VMEM((_G, _C), jnp.bfloat16),
            pltpu.VMEM((_G, _C), jnp.bfloat16),
            pltpu.VMEM((_G, _C), jnp.bfloat16),
            pltpu.VMEM((_G, _C), jnp.float32),
            pltpu.VMEM((_G, _C), jnp.float32),
            pltpu.VMEM((_G, _C), jnp.float32),
            pltpu.VMEM((_G, _C), jnp.float32),
            pltpu.VMEM((_G, _C), jnp.bfloat16),
            pltpu.SemaphoreType.DMA,
            pltpu.SemaphoreType.DMA,
            pltpu.SemaphoreType.DMA,
            pltpu.SemaphoreType.DMA,
            pltpu.SemaphoreType.DMA,
            pltpu.SemaphoreType.DMA,
        ],
        compiler_params=pltpu.CompilerParams(use_tc_tiling_on_sc=False,
                                             needs_layout_passes=False),
    )


def kernel(x, edge_index, edge_weight, W1, b1, W2, b2):
    dst = edge_index[0].reshape(_NW, _NCHUNK, _G)
    src = edge_index[1].reshape(_NW, _NCHUNK, _G)
    w = edge_weight.reshape(_NW, _NCHUNK, _G)
    zeros = jnp.zeros((_RPT, _C), jnp.float32)

    mlp = pl.pallas_call(
        _mlp_block,
        grid=(_N // _ROWBLK,),
        in_specs=[
            pl.BlockSpec((_ROWBLK, _D), lambda i: (i, 0)),
            pl.BlockSpec((_D, _H), lambda i: (0, 0)),
            pl.BlockSpec((1, _H), lambda i: (0, 0)),
            pl.BlockSpec((_H, _C), lambda i: (0, 0)),
            pl.BlockSpec((1, _C), lambda i: (0, 0)),
        ],
        out_specs=[
            pl.BlockSpec((_ROWBLK, _C), lambda i: (i, 0)),
            pl.BlockSpec((_ROWBLK, _C), lambda i: (i, 0)),
        ],
        out_shape=[
            jax.ShapeDtypeStruct((_N, _C), jnp.float32),
            jax.ShapeDtypeStruct((_NP, _C), jnp.float32),
        ],
    )
    z, z9 = mlp(x, W1, b1.reshape(1, _H), W2, b2.reshape(1, _C))

    step = _make_prop_step()

    final = pl.pallas_call(
        _final_block,
        grid=(_N // _ROWBLK,),
        in_specs=[
            pl.BlockSpec((1, _ROWBLK, _C), lambda i: (0, i, 0)),
            pl.BlockSpec((1, _ROWBLK, _C), lambda i: (1, i, 0)),
        ],
        out_specs=pl.BlockSpec((_ROWBLK, _C), lambda i: (i, 0)),
        out_shape=jax.ShapeDtypeStruct((_N, _C), jnp.float32),
    )

    part = jnp.zeros((_NC, _NP, _C), jnp.float32)
    part = part.at[0, :_N].set(z * (1.0 / (1.0 - _ALPHA)))
    for _ in range(_K):
        part = step(part, z9, src, dst, w, zeros)
    return final(part, part)


# speedup vs baseline: 1.2815x; 1.1642x over previous
import jax
import jax.numpy as jnp
from jax import lax
from jax.experimental import pallas as pl
from jax.experimental.pallas import tpu as pltpu
from jax.experimental.pallas import tpu_sc as plsc

_N = 10000
_E = 320000
_D = 128
_H = 64
_C = 64
_ALPHA = 0.1
_K = 10

_NC = 2
_NS = 16
_NW = _NC * _NS
_EPT = _E // _NW
_G = 80
_NCHUNK = _EPT // _G
_NP = 10112
_RPT = _NP // _NS
_SB = 1000

_ROWBLK = 1000


def _mlp_block(x_ref, w1_ref, b1_ref, w2_ref, b2_ref, z_ref, z9_ref):
    h = jnp.dot(x_ref[...], w1_ref[...], preferred_element_type=jnp.float32)
    h = jnp.maximum(h + b1_ref[...], 0.0)
    z = jnp.dot(h, w2_ref[...], preferred_element_type=jnp.float32) + b2_ref[...]
    z_ref[...] = z
    z9_ref[...] = z * (_ALPHA / (1.0 - _ALPHA))


def _final_block(a0_ref, a1_ref, o_ref):
    xk = (1.0 - _ALPHA) * (a0_ref[0] + a1_ref[0])
    m = jnp.max(xk, axis=1, keepdims=True)
    s = jnp.sum(jnp.exp(xk - m), axis=1, keepdims=True)
    o_ref[...] = (xk - m) - jnp.log(s)


def _prop_step_body(pp_hbm, z9_hbm, src_hbm, dst_hbm, w_hbm, zeros_hbm,
                    part_hbm, acc, xk_s, src_v, dst_v, w_v,
                    rows0, rows1, sbuf0, sbuf1, gsem0, gsem1, ssem0, ssem1):
    cid = lax.axis_index("c")
    sid = lax.axis_index("s")
    gid = cid * _NS + sid

    @pl.when(cid == 0)
    def _():
        pltpu.async_copy(z9_hbm.at[pl.ds(sid * _RPT, _RPT)],
                         acc.at[pl.ds(sid * _RPT, _RPT)], ssem0)

    @pl.when(cid == 1)
    def _():
        pltpu.async_copy(zeros_hbm, acc.at[pl.ds(sid * _RPT, _RPT)], ssem0)

    pltpu.async_copy(src_hbm.at[gid], src_v, ssem1)
    pltpu.async_copy(dst_hbm.at[gid], dst_v, gsem0)
    pltpu.async_copy(w_hbm.at[gid], w_v, gsem1)

    def _drain_slabs():
        pltpu.make_async_copy(dst_hbm.at[gid], dst_v, gsem0).wait()
        pltpu.make_async_copy(w_hbm.at[gid], w_v, gsem1).wait()

    def _drain_init():
        pltpu.make_async_copy(zeros_hbm, acc.at[pl.ds(sid * _RPT, _RPT)],
                              ssem0).wait()
        pltpu.make_async_copy(src_hbm.at[gid], src_v, ssem1).wait()

    _rounds = [(sid * _SB + b * _G, _G) for b in range(_SB // _G)]
    _rounds.append((sid * _SB + (_SB // _G) * _G, _SB % _G))
    _pairs = ((rows0, rows1, gsem0, gsem1), (sbuf0, sbuf1, ssem0, ssem1))

    def _issue(r, pa, pb, sa, sb):
        base, nrows = _rounds[r]
        pltpu.async_copy(pp_hbm.at[0, pl.ds(base, nrows)],
                         pa.at[pl.ds(0, nrows)], sa)
        pltpu.async_copy(pp_hbm.at[1, pl.ds(base, nrows)],
                         pb.at[pl.ds(0, nrows)], sb)

    def _finish(r, pa, pb, sa, sb):
        base, nrows = _rounds[r]
        pltpu.make_async_copy(pp_hbm.at[0, pl.ds(base, nrows)],
                              pa.at[pl.ds(0, nrows)], sa).wait()
        pltpu.make_async_copy(pp_hbm.at[1, pl.ds(base, nrows)],
                              pb.at[pl.ds(0, nrows)], sb).wait()

        def rowloop(rr, c2):
            for cpart in range(_C // 16):
                sl = pl.ds(cpart * 16, 16)
                pa[rr, sl] = (1.0 - _ALPHA) * (pa[rr, sl] + pb[rr, sl])
            return c2

        lax.fori_loop(0, nrows, rowloop, 0)
        pltpu.sync_copy(pa.at[pl.ds(0, nrows)], xk_s.at[pl.ds(base, nrows)])

    @pl.when(sid < 10)
    def _():
        _drain_slabs()
        _issue(0, *_pairs[0])
        for r in range(len(_rounds)):
            cur = _pairs[r % 2]
            if r + 1 < len(_rounds):
                if r == 0:
                    _drain_init()
                _issue(r + 1, *_pairs[(r + 1) % 2])
            _finish(r, *cur)

    @pl.when(sid >= 10)
    def _():
        _drain_slabs()
        _drain_init()

    plsc.subcore_barrier()

    rows = (rows0, rows1)
    sbufs = (sbuf0, sbuf1)
    gsems = (gsem0, gsem1)
    ssems = (ssem0, ssem1)

    def scale(src_buf, dst_buf, j):
        for r in range(_G // 16):
            w16 = w_v[j, pl.ds(r * 16, 16)]
            for e in range(16):
                wv = w16[e]
                row = r * 16 + e
                for cpart in range(_C // 16):
                    sl = pl.ds(cpart * 16, 16)
                    dst_buf[row, sl] = src_buf[row, sl] * wv

    pltpu.async_copy(xk_s.at[src_v.at[0]], rows0, gsem0)

    def body(t, carry):
        for p in range(2):
            i = 2 * t + p
            buf, sbuf = rows[p], sbufs[p]

            @pl.when(i < _NCHUNK)
            def _():
                pltpu.make_async_copy(xk_s.at[src_v.at[i]], buf,
                                      gsems[p]).wait()

                @pl.when(i < _NCHUNK - 1)
                def _():
                    pltpu.async_copy(xk_s.at[src_v.at[i + 1]], rows[1 - p],
                                     gsems[1 - p])

                @pl.when(i >= 2)
                def _():
                    pltpu.make_async_copy(sbuf, acc.at[dst_v.at[0]],
                                          ssems[p]).wait()

                scale(buf, sbuf, i)
                pltpu.async_copy(sbuf, acc.at[dst_v.at[i]], ssems[p],
                                 add=True)
        return carry

    lax.fori_loop(0, (_NCHUNK + 1) // 2, body, 0)
    pltpu.make_async_copy(sbuf1, acc.at[dst_v.at[0]], ssem1).wait()
    pltpu.make_async_copy(sbuf0, acc.at[dst_v.at[0]], ssem0).wait()

    plsc.subcore_barrier()
    pltpu.sync_copy(acc.at[pl.ds(sid * _RPT, _RPT)],
                    part_hbm.at[cid, pl.ds(sid * _RPT, _RPT)])


def _make_prop_step():
    mesh = plsc.VectorSubcoreMesh(core_axis_name="c", subcore_axis_name="s")
    return pl.kernel(
        _prop_step_body,
        out_type=jax.ShapeDtypeStruct((_NC, _NP, _C), jnp.float32),
        mesh=mesh,
        scratch_types=[
            pltpu.VMEM_SHARED((_NP, _C), jnp.float32),
            pltpu.VMEM_SHARED((_N, _C), jnp.float32),
            pltpu.VMEM((_NCHUNK, _G), jnp.int32),
            pltpu.VMEM((_NCHUNK, _G), jnp.int32),
            pltpu.VMEM((_NCHUNK, _G), jnp.float32),
            pltpu.VMEM((_G, _C), jnp.float32),
            pltpu.VMEM((_G, _C), jnp.float32),
            pltpu.VMEM((_G, _C), jnp.float32),
            pltpu.VMEM((_G, _C), jnp.float32),
            pltpu.SemaphoreType.DMA,
            pltpu.SemaphoreType.DMA,
            pltpu.SemaphoreType.DMA,
            pltpu.SemaphoreType.DMA,
        ],
        compiler_params=pltpu.CompilerParams(use_tc_tiling_on_sc=False),
    )


def kernel(x, edge_index, edge_weight, W1, b1, W2, b2):
    dst = edge_index[0].reshape(_NW, _NCHUNK, _G)
    src = edge_index[1].reshape(_NW, _NCHUNK, _G)
    w = edge_weight.reshape(_NW, _NCHUNK, _G)
    zeros = jnp.zeros((_RPT, _C), jnp.float32)

    mlp = pl.pallas_call(
        _mlp_block,
        grid=(_N // _ROWBLK,),
        in_specs=[
            pl.BlockSpec((_ROWBLK, _D), lambda i: (i, 0)),
            pl.BlockSpec((_D, _H), lambda i: (0, 0)),
            pl.BlockSpec((1, _H), lambda i: (0, 0)),
            pl.BlockSpec((_H, _C), lambda i: (0, 0)),
            pl.BlockSpec((1, _C), lambda i: (0, 0)),
        ],
        out_specs=[
            pl.BlockSpec((_ROWBLK, _C), lambda i: (i, 0)),
            pl.BlockSpec((_ROWBLK, _C), lambda i: (i, 0)),
        ],
        out_shape=[
            jax.ShapeDtypeStruct((_N, _C), jnp.float32),
            jax.ShapeDtypeStruct((_NP, _C), jnp.float32),
        ],
    )
    z, z9 = mlp(x, W1, b1.reshape(1, _H), W2, b2.reshape(1, _C))

    step = _make_prop_step()

    final = pl.pallas_call(
        _final_block,
        grid=(_N // _ROWBLK,),
        in_specs=[
            pl.BlockSpec((1, _ROWBLK, _C), lambda i: (0, i, 0)),
            pl.BlockSpec((1, _ROWBLK, _C), lambda i: (1, i, 0)),
        ],
        out_specs=pl.BlockSpec((_ROWBLK, _C), lambda i: (i, 0)),
        out_shape=jax.ShapeDtypeStruct((_N, _C), jnp.float32),
    )

    part = jnp.zeros((_NC, _NP, _C), jnp.float32)
    part = part.at[0, :_N].set(z * (1.0 / (1.0 - _ALPHA)))
    for _ in range(_K):
        part = step(part, z9, src, dst, w, zeros)
    return final(part, part)
